# baseline (speedup 1.0000x reference)
import functools
import jax
import jax.numpy as jnp
from jax import lax
from jax.experimental import pallas as pl
from jax.experimental.pallas import tpu as pltpu
from jax.experimental.pallas import tpu_sc as plsc

_NC, _NS, _L = 2, 16, 16


def kernel(x):
    B, H, W, C = x.shape
    HO, WO = H // 2, W // 2
    RO = B * HO
    NW = _NC * _NS
    RPW = RO // NW
    xt = jnp.transpose(x, (0, 1, 3, 2))
    mesh = plsc.VectorSubcoreMesh(
        core_axis_name="c", subcore_axis_name="s",
        num_cores=_NC, num_subcores=_NS)

    @functools.partial(
        pl.kernel,
        out_type=jax.ShapeDtypeStruct((B, HO, C, WO), x.dtype),
        mesh=mesh,
        scratch_types=[
            pltpu.VMEM((2, C, W), jnp.float32),
            pltpu.VMEM((C, WO), jnp.float32),
        ],
    )
    def k(x_hbm, o_hbm, buf, obuf):
        wid = lax.axis_index("s") * _NC + lax.axis_index("c")
        lane = lax.iota(jnp.int32, _L)
        idx_e = (lane * 2) % _L
        idx_o = idx_e + 1
        lo = lane < (_L // 2)

        gdn = lax.GatherDimensionNumbers(
            offset_dims=(), collapsed_slice_dims=(0,), start_index_map=(0,))

        def dg(v, idx):
            return lax.gather(
                v, idx[:, None], gdn, (1,),
                mode=lax.GatherScatterMode.PROMISE_IN_BOUNDS)

        def deint(v0, v1, idx):
            return jnp.where(lo, dg(v0, idx), dg(v1, idx))

        def row_body(r, carry):
            o = wid * RPW + r
            b = o // HO
            i = o % HO
            pltpu.sync_copy(x_hbm.at[b, 2 * i], buf.at[0])
            pltpu.sync_copy(x_hbm.at[b, 2 * i + 1], buf.at[1])

            for jv in range(WO // _L):
                w0 = jv * 2 * _L

                def c_body(c, carry3, w0=w0, jv=jv):
                    v00 = jnp.maximum(buf[0, c, pl.ds(w0, _L)], 0.0)
                    v01 = jnp.maximum(buf[0, c, pl.ds(w0 + _L, _L)], 0.0)
                    v10 = jnp.maximum(buf[1, c, pl.ds(w0, _L)], 0.0)
                    v11 = jnp.maximum(buf[1, c, pl.ds(w0 + _L, _L)], 0.0)
                    r0 = v00 + v10
                    r1 = v01 + v11
                    q0 = v00 * v00 + v10 * v10
                    q1 = v01 * v01 + v11 * v11
                    s = deint(r0, r1, idx_e) + deint(r0, r1, idx_o)
                    s2 = deint(q0, q1, idx_e) + deint(q0, q1, idx_o)
                    out = s2 / jnp.maximum(s, 1e-30)
                    obuf[c, pl.ds(jv * _L, _L)] = out
                    return carry3

                lax.fori_loop(0, C, c_body, 0, unroll=2)

            pltpu.sync_copy(obuf, o_hbm.at[b, i])
            return carry

        lax.fori_loop(0, RPW, row_body, 0)

    ot = k(xt)
    return jnp.transpose(ot, (0, 1, 3, 2))


# speedup vs baseline: 3.8141x; 3.8141x over previous
import functools
import jax
import jax.numpy as jnp
from jax import lax
from jax.experimental import pallas as pl
from jax.experimental.pallas import tpu as pltpu
from jax.experimental.pallas import tpu_sc as plsc

_NC, _NS, _L = 2, 16, 16
_B_TC = 14
_KH = 28


def _pool_body(x_ref, o_ref):
    KH, C, WO = o_ref.shape[1:]
    W = 2 * WO
    y = jnp.maximum(x_ref[0], 0.0)
    yr = y.reshape(KH, 2, C, W)
    t = yr[:, 0]
    b = yr[:, 1]
    r = t + b
    r2 = t * t + b * b
    u = lax.broadcasted_iota(jnp.int32, (W, WO), 0)
    j = lax.broadcasted_iota(jnp.int32, (W, WO), 1)
    q = jnp.where(u // 2 == j, 1.0, 0.0).astype(jnp.bfloat16)
    dn = (((2,), (0,)), ((), ()))

    def pair_sum(a):
        hi = a.astype(jnp.bfloat16)
        lo = (a - hi.astype(jnp.float32)).astype(jnp.bfloat16)
        d = lambda m: lax.dot_general(m, q, dn,
                                      preferred_element_type=jnp.float32)
        return d(hi) + d(lo)

    s = pair_sum(r)
    s2 = pair_sum(r2)
    o_ref[0] = jnp.where(s > 0, s2 / jnp.where(s > 0, s, 1.0), 0.0)


def _tc_part(xt, B, H, W, C):
    HO, WO = H // 2, W // 2
    grid = (_B_TC, HO // _KH)
    return pl.pallas_call(
        _pool_body,
        grid=grid,
        in_specs=[pl.BlockSpec((1, 2 * _KH, C, W), lambda b, i: (b, i, 0, 0))],
        out_specs=pl.BlockSpec((1, _KH, C, WO), lambda b, i: (b, i, 0, 0)),
        out_shape=jax.ShapeDtypeStruct((_B_TC, HO, C, WO), xt.dtype),
    )(xt)


def _sc_part(xt, B, H, W, C):
    HO, WO = H // 2, W // 2
    B_SC = B - _B_TC
    RO = B_SC * HO
    NW = _NC * _NS
    RPW = RO // NW
    mesh = plsc.VectorSubcoreMesh(
        core_axis_name="c", subcore_axis_name="s",
        num_cores=_NC, num_subcores=_NS)

    @functools.partial(
        pl.kernel,
        out_type=jax.ShapeDtypeStruct((B_SC, HO, C, WO), xt.dtype),
        mesh=mesh,
        scratch_types=[
            pltpu.VMEM((2, C, W), jnp.float32),
            pltpu.VMEM((C, WO), jnp.float32),
        ],
    )
    def k(x_hbm, o_hbm, buf, obuf):
        wid = lax.axis_index("s") * _NC + lax.axis_index("c")
        lane = lax.iota(jnp.int32, _L)
        idx_e = (lane * 2) % _L
        idx_o = idx_e + 1
        lo = lane < (_L // 2)

        gdn = lax.GatherDimensionNumbers(
            offset_dims=(), collapsed_slice_dims=(0,), start_index_map=(0,))

        def dg(v, idx):
            return lax.gather(
                v, idx[:, None], gdn, (1,),
                mode=lax.GatherScatterMode.PROMISE_IN_BOUNDS)

        def deint(v0, v1, idx):
            return jnp.where(lo, dg(v0, idx), dg(v1, idx))

        def row_body(r, carry):
            o = wid * RPW + r
            b = o // HO
            i = o % HO
            pltpu.sync_copy(x_hbm.at[_B_TC + b, 2 * i], buf.at[0])
            pltpu.sync_copy(x_hbm.at[_B_TC + b, 2 * i + 1], buf.at[1])

            for jv in range(WO // _L):
                w0 = jv * 2 * _L

                def c_body(c, carry3, w0=w0, jv=jv):
                    v00 = jnp.maximum(buf[0, c, pl.ds(w0, _L)], 0.0)
                    v01 = jnp.maximum(buf[0, c, pl.ds(w0 + _L, _L)], 0.0)
                    v10 = jnp.maximum(buf[1, c, pl.ds(w0, _L)], 0.0)
                    v11 = jnp.maximum(buf[1, c, pl.ds(w0 + _L, _L)], 0.0)
                    r0 = v00 + v10
                    r1 = v01 + v11
                    q0 = v00 * v00 + v10 * v10
                    q1 = v01 * v01 + v11 * v11
                    s = deint(r0, r1, idx_e) + deint(r0, r1, idx_o)
                    s2 = deint(q0, q1, idx_e) + deint(q0, q1, idx_o)
                    out = s2 / jnp.maximum(s, 1e-30)
                    obuf[c, pl.ds(jv * _L, _L)] = out
                    return carry3

                lax.fori_loop(0, C, c_body, 0, unroll=2)

            pltpu.sync_copy(obuf, o_hbm.at[b, i])
            return carry

        lax.fori_loop(0, RPW, row_body, 0)

    return k(xt)


def kernel(x):
    B, H, W, C = x.shape
    xt = jnp.transpose(x, (0, 1, 3, 2))
    ot_sc = _sc_part(xt, B, H, W, C)
    ot_tc = _tc_part(xt, B, H, W, C)
    ot = jnp.concatenate([ot_tc, ot_sc], axis=0)
    return jnp.transpose(ot, (0, 1, 3, 2))


# speedup vs baseline: 4.0408x; 1.0594x over previous
import functools
import jax
import jax.numpy as jnp
from jax import lax
from jax.experimental import pallas as pl
from jax.experimental.pallas import tpu as pltpu
from jax.experimental.pallas import tpu_sc as plsc

_NC, _NS, _L = 2, 16, 16
_B_TC = 14
_KH = 56


def _pool_body(x_ref, o_ref):
    KH, C, WO = o_ref.shape[1:]
    W = 2 * WO
    y = jnp.maximum(x_ref[0], 0.0)
    yr = y.reshape(KH, 2, C, W)
    t = yr[:, 0]
    b = yr[:, 1]
    r = t + b
    r2 = t * t + b * b
    u = lax.broadcasted_iota(jnp.int32, (W, WO), 0)
    j = lax.broadcasted_iota(jnp.int32, (W, WO), 1)
    q = jnp.where(u // 2 == j, 1.0, 0.0).astype(jnp.bfloat16)
    dn = (((2,), (0,)), ((), ()))

    def pair_sum(a):
        hi = a.astype(jnp.bfloat16)
        lo = (a - hi.astype(jnp.float32)).astype(jnp.bfloat16)
        d = lambda m: lax.dot_general(m, q, dn,
                                      preferred_element_type=jnp.float32)
        return d(hi) + d(lo)

    s = pair_sum(r)
    s2 = pair_sum(r2)
    o_ref[0] = jnp.where(s > 0, s2 / jnp.where(s > 0, s, 1.0), 0.0)


def _tc_part(xt, B, H, W, C):
    HO, WO = H // 2, W // 2
    grid = (_B_TC, HO // _KH)
    return pl.pallas_call(
        _pool_body,
        grid=grid,
        in_specs=[pl.BlockSpec((1, 2 * _KH, C, W), lambda b, i: (b, i, 0, 0))],
        out_specs=pl.BlockSpec((1, _KH, C, WO), lambda b, i: (b, i, 0, 0)),
        out_shape=jax.ShapeDtypeStruct((_B_TC, HO, C, WO), xt.dtype),
    )(xt)


def _sc_part(xt, B, H, W, C):
    HO, WO = H // 2, W // 2
    B_SC = B - _B_TC
    RO = B_SC * HO
    NW = _NC * _NS
    RPW = RO // NW
    mesh = plsc.VectorSubcoreMesh(
        core_axis_name="c", subcore_axis_name="s",
        num_cores=_NC, num_subcores=_NS)

    @functools.partial(
        pl.kernel,
        out_type=jax.ShapeDtypeStruct((B_SC, HO, C, WO), xt.dtype),
        mesh=mesh,
        scratch_types=[
            pltpu.VMEM((2, C, W), jnp.float32),
            pltpu.VMEM((C, WO), jnp.float32),
        ],
    )
    def k(x_hbm, o_hbm, buf, obuf):
        wid = lax.axis_index("s") * _NC + lax.axis_index("c")
        lane = lax.iota(jnp.int32, _L)
        idx_e = (lane * 2) % _L
        idx_o = idx_e + 1
        lo = lane < (_L // 2)

        gdn = lax.GatherDimensionNumbers(
            offset_dims=(), collapsed_slice_dims=(0,), start_index_map=(0,))

        def dg(v, idx):
            return lax.gather(
                v, idx[:, None], gdn, (1,),
                mode=lax.GatherScatterMode.PROMISE_IN_BOUNDS)

        def deint(v0, v1, idx):
            return jnp.where(lo, dg(v0, idx), dg(v1, idx))

        def row_body(r, carry):
            o = wid * RPW + r
            b = o // HO
            i = o % HO
            pltpu.sync_copy(x_hbm.at[_B_TC + b, 2 * i], buf.at[0])
            pltpu.sync_copy(x_hbm.at[_B_TC + b, 2 * i + 1], buf.at[1])

            for jv in range(WO // _L):
                w0 = jv * 2 * _L

                def c_body(c, carry3, w0=w0, jv=jv):
                    v00 = jnp.maximum(buf[0, c, pl.ds(w0, _L)], 0.0)
                    v01 = jnp.maximum(buf[0, c, pl.ds(w0 + _L, _L)], 0.0)
                    v10 = jnp.maximum(buf[1, c, pl.ds(w0, _L)], 0.0)
                    v11 = jnp.maximum(buf[1, c, pl.ds(w0 + _L, _L)], 0.0)
                    r0 = v00 + v10
                    r1 = v01 + v11
                    q0 = v00 * v00 + v10 * v10
                    q1 = v01 * v01 + v11 * v11
                    s = deint(r0, r1, idx_e) + deint(r0, r1, idx_o)
                    s2 = deint(q0, q1, idx_e) + deint(q0, q1, idx_o)
                    out = s2 / jnp.maximum(s, 1e-30)
                    obuf[c, pl.ds(jv * _L, _L)] = out
                    return carry3

                lax.fori_loop(0, C, c_body, 0, unroll=2)

            pltpu.sync_copy(obuf, o_hbm.at[b, i])
            return carry

        lax.fori_loop(0, RPW, row_body, 0)

    return k(xt)


def kernel(x):
    B, H, W, C = x.shape
    xt = jnp.transpose(x, (0, 1, 3, 2))
    ot_sc = _sc_part(xt, B, H, W, C)
    ot_tc = _tc_part(xt, B, H, W, C)
    ot = jnp.concatenate([ot_tc, ot_sc], axis=0)
    return jnp.transpose(ot, (0, 1, 3, 2))


# speedup vs baseline: 5.1163x; 1.2662x over previous
import functools
import jax
import jax.numpy as jnp
from jax import lax
from jax.experimental import pallas as pl
from jax.experimental.pallas import tpu as pltpu
from jax.experimental.pallas import tpu_sc as plsc

_NC, _NS, _L = 2, 16, 16
_B_TC = 14
_KH = 56


def _pool_body(x_ref, o_ref):
    KH, C, WO = o_ref.shape[1:]
    W = 2 * WO
    y = jnp.maximum(x_ref[0], 0.0)
    yr = y.reshape(KH, 2, C, W)
    t = yr[:, 0]
    b = yr[:, 1]
    r = t + b
    r2 = t * t + b * b
    u = lax.broadcasted_iota(jnp.int32, (W, WO), 0)
    j = lax.broadcasted_iota(jnp.int32, (W, WO), 1)
    q = jnp.where(u // 2 == j, 1.0, 0.0).astype(jnp.bfloat16)
    dn = (((2,), (0,)), ((), ()))

    def pair_sum(a):
        hi = a.astype(jnp.bfloat16)
        lo = (a - hi.astype(jnp.float32)).astype(jnp.bfloat16)
        d = lambda m: lax.dot_general(m, q, dn,
                                      preferred_element_type=jnp.float32)
        return d(hi) + d(lo)

    s = pair_sum(r)
    s2 = pair_sum(r2)
    o_ref[0] = jnp.where(s > 0, s2 / jnp.where(s > 0, s, 1.0), 0.0)


def _tc_part(xt, B, H, W, C):
    HO, WO = H // 2, W // 2
    grid = (_B_TC, HO // _KH)
    return pl.pallas_call(
        _pool_body,
        grid=grid,
        in_specs=[pl.BlockSpec((1, 2 * _KH, C, W), lambda b, i: (b, i, 0, 0))],
        out_specs=pl.BlockSpec((1, _KH, C, WO), lambda b, i: (b, i, 0, 0)),
        out_shape=jax.ShapeDtypeStruct((B, HO, C, WO), xt.dtype),
    )(xt)


def _patch_body(big_ref, sc_ref, o_ref):
    o_ref[...] = sc_ref[...]


def _patch(big, ot_sc):
    B, HO, C, WO = big.shape
    B_SC = ot_sc.shape[0]
    grid = (B_SC, HO // _KH)
    return pl.pallas_call(
        _patch_body,
        grid=grid,
        in_specs=[
            pl.BlockSpec(memory_space=pl.ANY),
            pl.BlockSpec((1, _KH, C, WO), lambda b, i: (b, i, 0, 0)),
        ],
        out_specs=pl.BlockSpec((1, _KH, C, WO),
                               lambda b, i: (b + _B_TC, i, 0, 0)),
        out_shape=jax.ShapeDtypeStruct((B, HO, C, WO), big.dtype),
        input_output_aliases={0: 0},
    )(big, ot_sc)


def _sc_part(xt, B, H, W, C):
    HO, WO = H // 2, W // 2
    B_SC = B - _B_TC
    RO = B_SC * HO
    NW = _NC * _NS
    RPW = RO // NW
    mesh = plsc.VectorSubcoreMesh(
        core_axis_name="c", subcore_axis_name="s",
        num_cores=_NC, num_subcores=_NS)

    @functools.partial(
        pl.kernel,
        out_type=jax.ShapeDtypeStruct((B_SC, HO, C, WO), xt.dtype),
        mesh=mesh,
        scratch_types=[
            pltpu.VMEM((2, C, W), jnp.float32),
            pltpu.VMEM((C, WO), jnp.float32),
        ],
    )
    def k(x_hbm, o_hbm, buf, obuf):
        wid = lax.axis_index("s") * _NC + lax.axis_index("c")
        lane = lax.iota(jnp.int32, _L)
        idx_e = (lane * 2) % _L
        idx_o = idx_e + 1
        lo = lane < (_L // 2)

        gdn = lax.GatherDimensionNumbers(
            offset_dims=(), collapsed_slice_dims=(0,), start_index_map=(0,))

        def dg(v, idx):
            return lax.gather(
                v, idx[:, None], gdn, (1,),
                mode=lax.GatherScatterMode.PROMISE_IN_BOUNDS)

        def deint(v0, v1, idx):
            return jnp.where(lo, dg(v0, idx), dg(v1, idx))

        def row_body(r, carry):
            o = wid * RPW + r
            b = o // HO
            i = o % HO
            pltpu.sync_copy(x_hbm.at[_B_TC + b, 2 * i], buf.at[0])
            pltpu.sync_copy(x_hbm.at[_B_TC + b, 2 * i + 1], buf.at[1])

            for jv in range(WO // _L):
                w0 = jv * 2 * _L

                def c_body(c, carry3, w0=w0, jv=jv):
                    v00 = jnp.maximum(buf[0, c, pl.ds(w0, _L)], 0.0)
                    v01 = jnp.maximum(buf[0, c, pl.ds(w0 + _L, _L)], 0.0)
                    v10 = jnp.maximum(buf[1, c, pl.ds(w0, _L)], 0.0)
                    v11 = jnp.maximum(buf[1, c, pl.ds(w0 + _L, _L)], 0.0)
                    r0 = v00 + v10
                    r1 = v01 + v11
                    q0 = v00 * v00 + v10 * v10
                    q1 = v01 * v01 + v11 * v11
                    s = deint(r0, r1, idx_e) + deint(r0, r1, idx_o)
                    s2 = deint(q0, q1, idx_e) + deint(q0, q1, idx_o)
                    out = s2 / jnp.maximum(s, 1e-30)
                    obuf[c, pl.ds(jv * _L, _L)] = out
                    return carry3

                lax.fori_loop(0, C, c_body, 0, unroll=2)

            pltpu.sync_copy(obuf, o_hbm.at[b, i])
            return carry

        lax.fori_loop(0, RPW, row_body, 0)

    return k(xt)


def kernel(x):
    B, H, W, C = x.shape
    xt = jnp.transpose(x, (0, 1, 3, 2))
    ot_sc = _sc_part(xt, B, H, W, C)
    big = _tc_part(xt, B, H, W, C)
    ot = _patch(big, ot_sc)
    return jnp.transpose(ot, (0, 1, 3, 2))
